# baseline (speedup 1.0000x reference)
import jax
import jax.numpy as jnp
from jax import lax
from jax.experimental import pallas as pl
from jax.experimental.pallas import tpu as pltpu
from jax.experimental.pallas import tpu_sc as plsc

B = 1024
L = 200
N_CAT = 26
VOCAB = 100000
EMB = 16
C_OUT = 16 + 8 + N_CAT * EMB
ROW = C_OUT * L
XN = 24 * L

NC = 2
NS = 16
NW = NC * NS

PAIRS_PER_W = (B * N_CAT) // NW
B_PER_W = B // NW

NSLOT = 8
SPLIT = 104
RPP = 2 * SPLIT
SB = 32
EL = EMB * L


SLAB = EL + XN


def _body(x_hbm, num_hbm, cat_hbm, tbl_hbm, out_hbm,
          staging, iabuf, rows, buf,
          g0, g1, g2, g3, g4, g5, g6, g7, osem0, osem1, rsem):
    wid = lax.axis_index("s") * NC + lax.axis_index("c")
    pair0 = wid * PAIRS_PER_W

    iota16 = lax.iota(jnp.int32, 16)
    iota_scaled = iota16 * L

    ia = tuple((iabuf.at[2 * s], iabuf.at[2 * s + 1]) for s in range(NSLOT))
    gsem = (g0, g1, g2, g3, g4, g5, g6, g7)
    osem = (osem0, osem1)

    for s in range(NSLOT):
        iabuf[2 * s + 1, pl.ds(88, 16)] = jnp.zeros((16,), jnp.int32)

    def gather_descs(s):
        return [
            pltpu.make_async_copy(
                tbl_hbm.at[ia[s][h]],
                rows.at[pl.ds(s * RPP + h * SPLIT, SPLIT)], gsem[s])
            for h in (0, 1)
        ]

    def adjust_and_fire(p, e, s):
        d = p - (p // N_CAT) * N_CAT
        off = jnp.full((16,), d * VOCAB, jnp.int32)
        for c in (0, 16, 32, 48, 64, 80, 88):
            iabuf[2 * s, pl.ds(c, 16)] = staging[e, pl.ds(c, 16)] + off
        for c in (0, 16, 32, 48, 64, 80):
            iabuf[2 * s + 1, pl.ds(c, 16)] = staging[e, pl.ds(SPLIT + c, 16)] + off
        for cp in gather_descs(s):
            cp.start()

    def wait_gathers(s):
        for cp in gather_descs(s):
            cp.wait()

    def out_desc(p, s):
        bb = p // N_CAT
        d = p - bb * N_CAT
        dst = bb * ROW + (24 + EMB * d) * L
        return pltpu.make_async_copy(buf.at[pl.ds((s % 2) * SLAB, EL)],
                                     out_hbm.at[pl.ds(dst, EL)], osem[s % 2])

    pltpu.sync_copy(cat_hbm.at[pl.ds(pair0, SB)], staging)
    for s in range(NSLOT):
        adjust_and_fire(pair0 + s, s, s)

    def relay_desc(bb, s):
        return pltpu.make_async_copy(buf.at[pl.ds(s * SLAB + EL, XN)],
                                     out_hbm.at[pl.ds(bb * ROW, XN)], rsem)

    for s in (0, 1):
        relay_desc(wid * B_PER_W + s, s).start()

    def relay(j2, _):
        for s in (0, 1):
            bb = wid * B_PER_W + 2 * j2 + s
            relay_desc(bb, s).wait()
            pltpu.sync_copy(x_hbm.at[pl.ds(bb * 16 * L, 16 * L)],
                            buf.at[pl.ds(s * SLAB + EL, 16 * L)])
            pltpu.sync_copy(num_hbm.at[pl.ds(bb * 8 * L, 8 * L)],
                            buf.at[pl.ds(s * SLAB + EL + 16 * L, 8 * L)])
            relay_desc(bb, s).start()
        return 0

    lax.fori_loop(0, B_PER_W // 2, relay, 0)
    for s in (0, 1):
        relay_desc(wid * B_PER_W + B_PER_W - 2 + s, s).wait()

    for s in (0, 1):
        out_desc(pair0 + s, s).start()

    def block(blk, _):
        wstart = pair0 + blk * SB + NSLOT
        pltpu.sync_copy(cat_hbm.at[pl.ds(wstart, SB)], staging)

        def step(k, _):
            for s in range(NSLOT):
                i = blk * SB + NSLOT * k + s
                p = pair0 + i
                wait_gathers(s)
                out_desc(p, s).wait()

                base_s = iota_scaled + (s % 2) * SLAB

                @plsc.parallel_loop(0, L, unroll=8)
                def tr(l, s=s, base_s=base_s):
                    vals = rows[s * RPP + l, :]
                    plsc.store_scatter(buf, [base_s + l], vals)

                out_desc(p, s).start()
                p2 = jnp.minimum(p + NSLOT, pair0 + PAIRS_PER_W - 1)
                adjust_and_fire(p2, p2 - wstart, s)
            return 0

        lax.fori_loop(0, SB // NSLOT, step, 0)
        return 0

    lax.fori_loop(0, PAIRS_PER_W // SB, block, 0)
    for s in (0, 1):
        out_desc(pair0 + PAIRS_PER_W - 2 + s, s).wait()
    for s in range(NSLOT):
        wait_gathers(s)


@jax.jit
def _run(x_flat, num_flat, cat2d, tbl_flat):
    mesh = plsc.VectorSubcoreMesh(core_axis_name="c", subcore_axis_name="s",
                                  num_cores=NC, num_subcores=NS)
    f = pl.kernel(
        _body,
        out_type=jax.ShapeDtypeStruct((B * ROW,), jnp.float32),
        mesh=mesh,
        compiler_params=pltpu.CompilerParams(use_tc_tiling_on_sc=False,
                                             needs_layout_passes=False),
        scratch_types=(
            [pltpu.VMEM((SB, L), jnp.int32)]
            + [pltpu.VMEM((2 * NSLOT, SPLIT), jnp.int32)]
            + [pltpu.VMEM((NSLOT * RPP, EMB), jnp.float32)]
            + [pltpu.VMEM((2 * (EL + XN),), jnp.float32)]
            + [pltpu.SemaphoreType.DMA] * (NSLOT + 3)
        ),
    )
    return f(x_flat, num_flat, cat2d, tbl_flat)


def kernel(x, num_feat, cat_feat, tables):
    cat2d = jnp.concatenate(
        [cat_feat.reshape(B * N_CAT, L), jnp.zeros((8, L), jnp.int32)])
    out = _run(x.reshape(-1), num_feat.reshape(-1), cat2d,
               tables.reshape(N_CAT * VOCAB, EMB))
    return out.reshape(B, C_OUT, L)


# speedup vs baseline: 1.0040x; 1.0040x over previous
import jax
import jax.numpy as jnp
from jax import lax
from jax.experimental import pallas as pl
from jax.experimental.pallas import tpu as pltpu
from jax.experimental.pallas import tpu_sc as plsc

B = 1024
L = 200
N_CAT = 26
VOCAB = 100000
EMB = 16
C_OUT = 16 + 8 + N_CAT * EMB
ROW = C_OUT * L
XN = 24 * L

NC = 2
NS = 16
NW = NC * NS

PAIRS_PER_W = (B * N_CAT) // NW
B_PER_W = B // NW

NSLOT = 8
SPLIT = 104
RPP = 2 * SPLIT
SB = 32
EL = EMB * L


SLAB = EL + XN


def _body(x_hbm, num_hbm, cat_hbm, tbl_hbm, out_hbm,
          staging, iabuf, rows, buf,
          g0, g1, g2, g3, g4, g5, g6, g7, osem0, osem1, rsem):
    wid = lax.axis_index("s") * NC + lax.axis_index("c")
    pair0 = wid * PAIRS_PER_W

    iota16 = lax.iota(jnp.int32, 16)
    iota_scaled = iota16 * L

    ia = tuple((iabuf.at[2 * s], iabuf.at[2 * s + 1]) for s in range(NSLOT))
    gsem = (g0, g1, g2, g3, g4, g5, g6, g7)
    osem = (osem0, osem1)

    for s in range(NSLOT):
        iabuf[2 * s + 1, pl.ds(88, 16)] = jnp.zeros((16,), jnp.int32)

    def gather_descs(s):
        return [
            pltpu.make_async_copy(
                tbl_hbm.at[ia[s][h]],
                rows.at[pl.ds(s * RPP + h * SPLIT, SPLIT)], gsem[s])
            for h in (0, 1)
        ]

    def adjust_and_fire(p, e, s):
        d = p - (p // N_CAT) * N_CAT
        off = jnp.full((16,), d * VOCAB, jnp.int32)
        for c in (0, 16, 32, 48, 64, 80, 88):
            iabuf[2 * s, pl.ds(c, 16)] = staging[e, pl.ds(c, 16)] + off
        for c in (0, 16, 32, 48, 64, 80):
            iabuf[2 * s + 1, pl.ds(c, 16)] = staging[e, pl.ds(SPLIT + c, 16)] + off
        for cp in gather_descs(s):
            cp.start()

    def wait_gathers(s):
        for cp in gather_descs(s):
            cp.wait()

    def out_desc(p, s):
        bb = p // N_CAT
        d = p - bb * N_CAT
        dst = bb * ROW + (24 + EMB * d) * L
        return pltpu.make_async_copy(buf.at[pl.ds((s % 2) * SLAB, EL)],
                                     out_hbm.at[pl.ds(dst, EL)], osem[s % 2])

    pltpu.sync_copy(cat_hbm.at[pl.ds(pair0, SB)], staging)
    for s in range(NSLOT):
        adjust_and_fire(pair0 + s, s, s)

    def relay_desc(bb, s):
        return pltpu.make_async_copy(buf.at[pl.ds(s * SLAB + EL, XN)],
                                     out_hbm.at[pl.ds(bb * ROW, XN)], rsem)

    for s in (0, 1):
        relay_desc(wid * B_PER_W + s, s).start()

    def relay(j2, _):
        for s in (0, 1):
            bb = wid * B_PER_W + 2 * j2 + s
            relay_desc(bb, s).wait()
            pltpu.sync_copy(x_hbm.at[pl.ds(bb * 16 * L, 16 * L)],
                            buf.at[pl.ds(s * SLAB + EL, 16 * L)])
            pltpu.sync_copy(num_hbm.at[pl.ds(bb * 8 * L, 8 * L)],
                            buf.at[pl.ds(s * SLAB + EL + 16 * L, 8 * L)])
            relay_desc(bb, s).start()
        return 0

    lax.fori_loop(0, B_PER_W // 2, relay, 0)
    for s in (0, 1):
        relay_desc(wid * B_PER_W + B_PER_W - 2 + s, s).wait()

    for s in (0, 1):
        out_desc(pair0 + s, s).start()

    def block(blk, _):
        wstart = pair0 + jnp.minimum(blk * SB + NSLOT, PAIRS_PER_W - SB)
        pltpu.sync_copy(cat_hbm.at[pl.ds(wstart, SB)], staging)

        def step(k, _):
            for s in range(NSLOT):
                i = blk * SB + NSLOT * k + s
                p = pair0 + i
                wait_gathers(s)
                out_desc(p, s).wait()

                base_s = iota_scaled + (s % 2) * SLAB

                @plsc.parallel_loop(0, L, unroll=8)
                def tr(l, s=s, base_s=base_s):
                    vals = rows[s * RPP + l, :]
                    plsc.store_scatter(buf, [base_s + l], vals)

                out_desc(p, s).start()
                p2 = jnp.minimum(p + NSLOT, pair0 + PAIRS_PER_W - 1)
                adjust_and_fire(p2, p2 - wstart, s)
            return 0

        lax.fori_loop(0, SB // NSLOT, step, 0)
        return 0

    lax.fori_loop(0, PAIRS_PER_W // SB, block, 0)
    for s in (0, 1):
        out_desc(pair0 + PAIRS_PER_W - 2 + s, s).wait()
    for s in range(NSLOT):
        wait_gathers(s)


@jax.jit
def _run(x_flat, num_flat, cat2d, tbl_flat):
    mesh = plsc.VectorSubcoreMesh(core_axis_name="c", subcore_axis_name="s",
                                  num_cores=NC, num_subcores=NS)
    f = pl.kernel(
        _body,
        out_type=jax.ShapeDtypeStruct((B * ROW,), jnp.float32),
        mesh=mesh,
        compiler_params=pltpu.CompilerParams(use_tc_tiling_on_sc=False,
                                             needs_layout_passes=False),
        scratch_types=(
            [pltpu.VMEM((SB, L), jnp.int32)]
            + [pltpu.VMEM((2 * NSLOT, SPLIT), jnp.int32)]
            + [pltpu.VMEM((NSLOT * RPP, EMB), jnp.float32)]
            + [pltpu.VMEM((2 * (EL + XN),), jnp.float32)]
            + [pltpu.SemaphoreType.DMA] * (NSLOT + 3)
        ),
    )
    return f(x_flat, num_flat, cat2d, tbl_flat)


def kernel(x, num_feat, cat_feat, tables):
    out = _run(x.reshape(-1), num_feat.reshape(-1),
               cat_feat.reshape(B * N_CAT, L),
               tables.reshape(N_CAT * VOCAB, EMB))
    return out.reshape(B, C_OUT, L)
